# baseline (speedup 1.0000x reference)
import functools

import jax
import jax.numpy as jnp
from jax import lax
from jax.experimental import pallas as pl
from jax.experimental.pallas import tpu as pltpu
from jax.experimental.pallas import tpu_sc as plsc

VOCAB = 1000000
DIM = 64
BATCH = 4096
HIST = 200

NC = 2
NS = 16
NW = NC * NS

B = BATCH * HIST
B_PER_W = B // NW
CHUNK = 512
NSTEP = B_PER_W // CHUNK


@functools.partial(
    pl.kernel,
    out_type=jax.ShapeDtypeStruct((B, DIM), jnp.float32),
    mesh=plsc.VectorSubcoreMesh(
        core_axis_name="c", subcore_axis_name="s", num_cores=NC, num_subcores=NS
    ),
    scratch_types=[
        pltpu.VMEM((CHUNK,), jnp.int32),
        pltpu.VMEM((CHUNK, DIM), jnp.float32),
        pltpu.SemaphoreType.DMA,
    ],
    compiler_params=pltpu.CompilerParams(use_tc_tiling_on_sc=False),
)
def _gather_kernel(idx_hbm, table_hbm, out_hbm, idx_v, rows_v, sem):
    wid = lax.axis_index("s") * NC + lax.axis_index("c")
    w_base = wid * B_PER_W

    def step(i, carry):
        base = w_base + i * CHUNK
        pltpu.sync_copy(idx_hbm.at[pl.ds(base, CHUNK)], idx_v)
        pltpu.async_copy(table_hbm.at[idx_v], rows_v, sem).wait()
        pltpu.sync_copy(rows_v, out_hbm.at[pl.ds(base, CHUNK)])
        return carry

    lax.fori_loop(0, NSTEP, step, 0)


def kernel(inputs, embed_weight):
    flat_idx = inputs.reshape((B,)).astype(jnp.int32)
    out = _gather_kernel(flat_idx, embed_weight)
    return out.reshape((BATCH, HIST, DIM))


# speedup vs baseline: 1.0422x; 1.0422x over previous
import functools

import jax
import jax.numpy as jnp
from jax import lax
from jax.experimental import pallas as pl
from jax.experimental.pallas import tpu as pltpu
from jax.experimental.pallas import tpu_sc as plsc

VOCAB = 1000000
DIM = 64
BATCH = 4096
HIST = 200

NC = 2
NS = 16
NW = NC * NS

B = BATCH * HIST
B_PER_W = B // NW
CHUNK = 512
NSTEP = B_PER_W // CHUNK
NBUF = 2


@functools.partial(
    pl.kernel,
    out_type=jax.ShapeDtypeStruct((B, DIM), jnp.float32),
    mesh=plsc.VectorSubcoreMesh(
        core_axis_name="c", subcore_axis_name="s", num_cores=NC, num_subcores=NS
    ),
    scratch_types=[
        pltpu.VMEM((B_PER_W,), jnp.int32),
        *[pltpu.VMEM((CHUNK, DIM), jnp.float32) for _ in range(NBUF)],
        *[pltpu.SemaphoreType.DMA for _ in range(2 * NBUF)],
    ],
    compiler_params=pltpu.CompilerParams(use_tc_tiling_on_sc=False),
)
def _gather_kernel(idx_hbm, table_hbm, out_hbm, idx_all, *bufs):
    rows = list(bufs[:NBUF])
    gsem = list(bufs[NBUF : 2 * NBUF])
    osem = list(bufs[2 * NBUF : 3 * NBUF])

    wid = lax.axis_index("s") * NC + lax.axis_index("c")
    w_base = wid * B_PER_W

    pltpu.sync_copy(idx_hbm.at[pl.ds(w_base, B_PER_W)], idx_all)

    def idx_slice(i):
        return idx_all.at[pl.ds(i * CHUNK, CHUNK)]

    def out_slice(i):
        return out_hbm.at[pl.ds(w_base + i * CHUNK, CHUNK)]

    def start_gather(i, b):
        pltpu.async_copy(table_hbm.at[idx_slice(i)], rows[b], gsem[b])

    def wait_gather(i, b):
        pltpu.make_async_copy(table_hbm.at[idx_slice(i)], rows[b], gsem[b]).wait()

    def start_store(i, b):
        pltpu.async_copy(rows[b], out_slice(i), osem[b])

    def wait_store(i, b):
        pltpu.make_async_copy(rows[b], out_slice(i), osem[b]).wait()

    start_gather(0, 0)

    def outer(g, carry):
        for b in range(NBUF):
            i = g * NBUF + b
            bn = (b + 1) % NBUF
            wait_gather(i, b)
            start_store(i, b)

            @pl.when(i + 1 < NSTEP)
            def _():
                @pl.when(i + 1 >= NBUF)
                def _():
                    wait_store(i + 1 - NBUF, bn)

                start_gather(i + 1, bn)

        return carry

    lax.fori_loop(0, NSTEP // NBUF, outer, 0)

    for b in range(NBUF):
        wait_store(NSTEP - NBUF + b, b)


def kernel(inputs, embed_weight):
    flat_idx = inputs.reshape((B,)).astype(jnp.int32)
    out = _gather_kernel(flat_idx, embed_weight)
    return out.reshape((BATCH, HIST, DIM))
